# Initial kernel scaffold; baseline (speedup 1.0000x reference)
import jax
import jax.numpy as jnp
from jax.experimental import pallas as pl


def kernel(occ_probs, occ_xyz, b_inds):
    raise NotImplementedError("write your pallas kernel here")


# speedup vs baseline: 3.4845x; 3.4845x over previous
import functools

import jax
import jax.numpy as jnp
from jax import lax
from jax.experimental import pallas as pl
from jax.experimental.pallas import tpu as pltpu, tpu_sc as plsc

I32 = jnp.int32
F32 = jnp.float32

_B = 4
_NZ, _NY, _NX = 20, 500, 440
_N = 2_000_000
_NBIN_TOT = _B * _NZ * _NY * _NX
_K = 12000
_PER_BATCH = _NZ * _NY * _NX

_NCH = 1000
_CH = 2000

_SZ = 1_280_000
_NPASS = 7
_TZ = _SZ // 16
_ZCH = 20000

_VBINS = 8192
_KSEL = _K + 1024
_CAPT = 4096
_CBUF = _CAPT + 64
_SCAP = 16384
_SH_CAP = 18432
_CH3 = 4400

_mesh = plsc.VectorSubcoreMesh(core_axis_name="c", subcore_axis_name="s")
_params = pltpu.CompilerParams(needs_layout_passes=False)


def _iota():
    return lax.iota(I32, 16)


def _zeros16():
    return lax.full((16,), 0, I32)


def _ones16():
    return lax.full((16,), 1, I32)


def _scal(vec, lane):
    return jnp.sum(jnp.where(_iota() == lane, vec, 0))


@functools.partial(
    pl.kernel,
    out_type=(
        jax.ShapeDtypeStruct((_N * 4,), I32),
        jax.ShapeDtypeStruct((_N,), I32),
    ),
    mesh=_mesh,
    compiler_params=_params,
    scratch_types=[
        pltpu.VMEM((_CH * 3,), F32),
        pltpu.VMEM((_CH,), I32),
        pltpu.VMEM((_CH * 4,), I32),
        pltpu.VMEM((_CH,), I32),
    ],
)
def _k1(xyz_hbm, b_hbm, vox_hbm, lin_hbm, v_in, v_b, v_out, v_lin):
    cid_c = lax.axis_index("c")
    sid = lax.axis_index("s")
    wid = cid_c * 16 + sid
    nch = 31 + jnp.where(wid < _NCH - 31 * 32, 1, 0)

    def chunk(j, carry):
        cid = wid + 32 * j
        base = cid * _CH
        pltpu.sync_copy(xyz_hbm.at[pl.ds(base * 3, _CH * 3)], v_in)
        pltpu.sync_copy(b_hbm.at[pl.ds(base, _CH)], v_b)

        def vec(i, carry2):
            q = i * 16 + _iota()
            ux = plsc.load_gather(v_in, [q * 3])
            uy = plsc.load_gather(v_in, [q * 3 + 1])
            uz = plsc.load_gather(v_in, [q * 3 + 2])
            cx = ((ux * 70.4 + 0.0) - 0.0) / 0.16
            cy = ((uy * 80.0 + -40.0) - -40.0) / 0.16
            cz = ((uz * 4.0 + -3.0) - -3.0) / 0.2
            cxi = jnp.clip(cx.astype(I32), 0, _NX - 1)
            cyi = jnp.clip(cy.astype(I32), 0, _NY - 1)
            czi = jnp.clip(cz.astype(I32), 0, _NZ - 1)
            bb = v_b[pl.ds(i * 16, 16)]
            lin = ((bb * _NZ + czi) * _NY + cyi) * _NX + cxi
            plsc.store_scatter(v_out, [q * 4], bb)
            plsc.store_scatter(v_out, [q * 4 + 1], czi)
            plsc.store_scatter(v_out, [q * 4 + 2], cyi)
            plsc.store_scatter(v_out, [q * 4 + 3], cxi)
            v_lin[pl.ds(i * 16, 16)] = lin
            return carry2

        lax.fori_loop(0, _CH // 16, vec, 0)
        pltpu.sync_copy(v_out, vox_hbm.at[pl.ds(base * 4, _CH * 4)])
        pltpu.sync_copy(v_lin, lin_hbm.at[pl.ds(base, _CH)])
        return carry

    lax.fori_loop(0, nch, chunk, 0)


@functools.partial(
    pl.kernel,
    out_type=jax.ShapeDtypeStruct((_NBIN_TOT,), I32),
    mesh=_mesh,
    compiler_params=_params,
    scratch_types=[
        pltpu.VMEM((_CH,), I32),
        pltpu.VMEM((_CH,), I32),
        pltpu.VMEM((_CH,), I32),
        pltpu.VMEM((_ZCH,), I32),
        pltpu.VMEM((_ZCH,), I32),
        pltpu.VMEM_SHARED((_SZ,), I32),
    ],
)
def _k2(lin_hbm, hist_hbm, v_lin, v_idx, v_one, v_zero, v_obuf, sh_bins):
    cid_c = lax.axis_index("c")
    sid = lax.axis_index("s")
    nch = 62 + jnp.where(sid < _NCH - 62 * 16, 1, 0)

    def fill(i, c):
        v_one[pl.ds(i * 16, 16)] = _ones16()
        return c

    lax.fori_loop(0, _CH // 16, fill, 0)

    def fillz(i, c):
        v_zero[pl.ds(i * 16, 16)] = _zeros16()
        return c

    lax.fori_loop(0, _ZCH // 16, fillz, 0)

    for p in range(_NPASS):
        base = (2 * p + cid_c) * _SZ
        for t in range(_TZ // _ZCH):
            pltpu.sync_copy(
                v_zero, sh_bins.at[pl.ds(sid * _TZ + t * _ZCH, _ZCH)])
        plsc.subcore_barrier()

        def chunk(j, carry):
            cid = sid + 16 * j
            pltpu.sync_copy(lin_hbm.at[pl.ds(cid * _CH, _CH)], v_lin)

            def vec(i, c2):
                loc = v_lin[pl.ds(i * 16, 16)] - base
                inb = (loc >= 0) & (loc < _SZ)
                v_idx[pl.ds(i * 16, 16)] = jnp.where(inb, loc, -1)
                return c2

            lax.fori_loop(0, _CH // 16, vec, 0)
            pltpu.sync_copy(
                v_one, sh_bins.at[plsc.Indices(v_idx, ignored_value=-1)],
                add=True)
            return carry

        lax.fori_loop(0, nch, chunk, 0)
        plsc.subcore_barrier()
        for t in range(_TZ // _ZCH):
            off = sid * _TZ + t * _ZCH

            @pl.when(base + off + _ZCH <= _NBIN_TOT)
            def _():
                pltpu.sync_copy(sh_bins.at[pl.ds(off, _ZCH)], v_obuf)
                pltpu.sync_copy(
                    v_obuf, hist_hbm.at[pl.ds(base + off, _ZCH)])

        plsc.subcore_barrier()


@functools.partial(
    pl.kernel,
    out_type=(
        jax.ShapeDtypeStruct((_B * _K,), F32),
        jax.ShapeDtypeStruct((_B * _K,), I32),
        jax.ShapeDtypeStruct((16,), I32),
    ),
    mesh=_mesh,
    compiler_params=_params,
    scratch_types=[
        pltpu.VMEM((_CH3,), F32),
        pltpu.VMEM((_VBINS,), I32),
        pltpu.VMEM((_VBINS,), I32),
        pltpu.VMEM((_CBUF,), F32),
        pltpu.VMEM((_CBUF,), I32),
        pltpu.VMEM((16,), I32),
        pltpu.VMEM((_SCAP,), F32),
        pltpu.VMEM((_SCAP,), I32),
        pltpu.VMEM((_SCAP,), I32),
        pltpu.VMEM((_SCAP,), I32),
        pltpu.VMEM((_SCAP,), I32),
        pltpu.VMEM((1024,), I32),
        pltpu.VMEM((1024,), I32),
        pltpu.VMEM_SHARED((16 * _VBINS,), I32),
        pltpu.VMEM_SHARED((2 * _SH_CAP,), F32),
        pltpu.VMEM_SHARED((2 * _SH_CAP,), I32),
        pltpu.VMEM_SHARED((48 * 16,), I32),
    ],
)
def _k3(occ_hbm, tv_hbm, ti_hbm, occ_hbm_out,
        v_buf, v_hist, v_acc, v_cv, v_ci, v_meta,
        s_fv, s_ka, s_ia, s_kb, s_ib, s_dh, s_do,
        sh_hist, sh_v, sh_i, sh_meta):
    cid_c = lax.axis_index("c")
    sid = lax.axis_index("s")
    lb = sid // 8
    m = sid % 8
    b = 2 * cid_c + lb
    dbase = b * _PER_BATCH + m * 550000
    io = _iota()

    def zh(i, c):
        v_hist[pl.ds(i * 16, 16)] = _zeros16()
        return c

    lax.fori_loop(0, _VBINS // 16, zh, 0)

    def p1chunk(j, cntv):
        pltpu.sync_copy(occ_hbm.at[pl.ds(dbase + j * _CH3, _CH3)], v_buf)

        def vec(i, cv):
            v = v_buf[pl.ds(i * 16, 16)]
            bn = jnp.minimum((v * float(_VBINS)).astype(I32), _VBINS - 1)
            plsc.addupdate_scatter(v_hist, [bn], _ones16())
            return cv + jnp.where(v > 0.5, 1, 0)

        return lax.fori_loop(0, _CH3 // 16, vec, cntv)

    cntv = lax.fori_loop(0, 550000 // _CH3, p1chunk, _zeros16())
    pltpu.sync_copy(v_hist, sh_hist.at[pl.ds(sid * _VBINS, _VBINS)])
    v_meta[...] = cntv
    pltpu.sync_copy(v_meta, sh_meta.at[pl.ds(sid * 16, 16)])
    plsc.subcore_barrier()

    @pl.when(m == 0)
    def _():
        def za(i, c):
            v_acc[pl.ds(i * 16, 16)] = _zeros16()
            return c

        lax.fori_loop(0, _VBINS // 16, za, 0)
        occv = _zeros16()
        for r in range(8):
            pltpu.sync_copy(sh_hist.at[pl.ds((lb * 8 + r) * _VBINS, _VBINS)], v_hist)

            def addh(i, c):
                v_acc[pl.ds(i * 16, 16)] = (
                    v_acc[pl.ds(i * 16, 16)] + v_hist[pl.ds(i * 16, 16)])
                return c

            lax.fori_loop(0, _VBINS // 16, addh, 0)
            pltpu.sync_copy(sh_meta.at[pl.ds((lb * 8 + r) * 16, 16)], v_meta)
            occv = occv + v_meta[...]
        occ_b = jnp.sum(occv)

        def scan(j, carry):
            total, cbin, found = carry
            jj = _VBINS // 16 - 1 - j
            h = v_acc[pl.ds(jj * 16, 16)]
            sfx = lax.rev(plsc.cumsum(lax.rev(h, (0,))), (0,))
            s_incl = total + sfx
            good = (s_incl >= _KSEL).astype(I32)
            ngood = jnp.sum(good)
            cand = jj * 16 + ngood - 1
            cbin = jnp.where(found == 0, jnp.where(ngood > 0, cand, cbin), cbin)
            found = jnp.where(ngood > 0, 1, found)
            return total + jnp.sum(h), cbin, found

        _, cbin, _ = lax.fori_loop(0, _VBINS // 16, scan, (0, 0, 0))
        v_meta[...] = jnp.where(io == 0, cbin, 0) + jnp.where(io == 1, occ_b, 0)
        pltpu.sync_copy(v_meta, sh_meta.at[pl.ds((32 + lb) * 16, 16)])

    plsc.subcore_barrier()

    @pl.when(sid == 0)
    def _():
        pltpu.sync_copy(sh_meta.at[pl.ds(32 * 16, 16)], v_meta)
        o0 = _scal(v_meta[...], 1)
        pltpu.sync_copy(sh_meta.at[pl.ds(33 * 16, 16)], v_meta)
        o1 = _scal(v_meta[...], 1)
        v_meta[...] = jnp.where(io == 0, o0, 0) + jnp.where(io == 1, o1, 0)
        pltpu.sync_copy(v_meta.at[pl.ds(0, 8)],
                        occ_hbm_out.at[pl.ds(8 * cid_c, 8)])

    pltpu.sync_copy(sh_meta.at[pl.ds((32 + lb) * 16, 16)], v_meta)
    cbin = _scal(v_meta[...], 0)

    def p3chunk(j, off):
        pltpu.sync_copy(occ_hbm.at[pl.ds(dbase + j * _CH3, _CH3)], v_buf)

        def vec(i, off2):
            v = v_buf[pl.ds(i * 16, 16)]
            bn = jnp.minimum((v * float(_VBINS)).astype(I32), _VBINS - 1)
            msk = bn >= cbin
            gidx = m * 550000 + j * _CH3 + i * 16 + io

            @pl.when(off2 <= _CAPT)
            def _():
                plsc.store_compressed(v_cv.at[pl.ds(off2, 16)], v, mask=msk)
                plsc.store_compressed(v_ci.at[pl.ds(off2, 16)], gidx, mask=msk)

            return jnp.minimum(off2 + jnp.sum(msk.astype(I32)), _CAPT + 16)

        return lax.fori_loop(0, _CH3 // 16, vec, off)

    off = lax.fori_loop(0, 550000 // _CH3, p3chunk, 0)
    npad = (-off) & 15
    padmask = io < npad
    plsc.store_compressed(v_cv.at[pl.ds(off, 16)],
                          lax.full((16,), -1.0, F32), mask=padmask)
    plsc.store_compressed(v_ci.at[pl.ds(off, 16)], _zeros16(), mask=padmask)
    offp = off + npad
    v_meta[...] = jnp.where(io == 0, off, 0) + jnp.where(io == 1, offp, 0)
    pltpu.sync_copy(v_meta, sh_meta.at[pl.ds((16 + sid) * 16, 16)])
    plsc.subcore_barrier()

    pref = 0
    n_real = 0
    n_pad = 0
    for r in range(8):
        pltpu.sync_copy(sh_meta.at[pl.ds((16 + lb * 8 + r) * 16, 16)], v_meta)
        cr = _scal(v_meta[...], 0)
        cp = _scal(v_meta[...], 1)
        pref = pref + jnp.where(r < m, cp, 0)
        n_real = n_real + cr
        n_pad = n_pad + cp
    mine = offp

    def pub(t, c):
        dst = pl.multiple_of(pref + t * 16, 16)

        @pl.when(dst <= _SH_CAP - 16)
        def _():
            pltpu.sync_copy(v_cv.at[pl.ds(t * 16, 16)],
                            sh_v.at[pl.ds(lb * _SH_CAP + dst, 16)])
            pltpu.sync_copy(v_ci.at[pl.ds(t * 16, 16)],
                            sh_i.at[pl.ds(lb * _SH_CAP + dst, 16)])

        return c

    lax.fori_loop(0, mine // 16, pub, 0)
    plsc.subcore_barrier()

    @pl.when(m == 0)
    def _():
        n_eff = jnp.minimum(n_pad, _SCAP)
        nv = n_eff // 16
        pltpu.sync_copy(sh_v.at[pl.ds(lb * _SH_CAP, _SCAP)], s_fv)
        pltpu.sync_copy(sh_i.at[pl.ds(lb * _SH_CAP, _SCAP)], s_ia)

        def keys(i, c):
            v = s_fv[pl.ds(i * 16, 16)]
            bits = plsc.bitcast(v, I32)
            key = jnp.where(bits < 0, 0x3FFFFFFF, 0x3FFFFFFE - bits)
            s_ka[pl.ds(i * 16, 16)] = key
            return c

        lax.fori_loop(0, nv, keys, 0)

        def radix_pass(shift, src_k, src_i, dst_k, dst_i):
            def zd(i, c):
                s_dh[pl.ds(i * 16, 16)] = _zeros16()
                return c

            lax.fori_loop(0, 64, zd, 0)

            def hist(i, c):
                d = (lax.shift_right_logical(src_k[pl.ds(i * 16, 16)], shift)
                     & 1023)
                cnt, last = plsc.scan_count(d)
                plsc.addupdate_scatter(s_dh, [d], cnt, mask=last)
                return c

            lax.fori_loop(0, nv, hist, 0)

            def pfx(i, tot):
                h = s_dh[pl.ds(i * 16, 16)]
                cs = plsc.cumsum(h)
                s_do[pl.ds(i * 16, 16)] = tot + cs - h
                return tot + jnp.sum(h)

            lax.fori_loop(0, 64, pfx, 0)

            def scat(i, c):
                k = src_k[pl.ds(i * 16, 16)]
                pv = src_i[pl.ds(i * 16, 16)]
                d = lax.shift_right_logical(k, shift) & 1023
                cnt, last = plsc.scan_count(d)
                pos = plsc.load_gather(s_do, [d]) + cnt - 1
                plsc.store_scatter(dst_k, [pos], k)
                plsc.store_scatter(dst_i, [pos], pv)
                plsc.addupdate_scatter(s_do, [d], cnt, mask=last)
                return c

            lax.fori_loop(0, nv, scat, 0)

        radix_pass(0, s_ka, s_ia, s_kb, s_ib)
        radix_pass(10, s_kb, s_ib, s_ka, s_ia)
        radix_pass(20, s_ka, s_ia, s_kb, s_ib)

        def emit(i, c):
            key = s_kb[pl.ds(i * 16, 16)]
            bits = 0x3FFFFFFE - key
            v = plsc.bitcast(bits, F32)
            s_fv[pl.ds(i * 16, 16)] = jnp.where(v > 0.5, v, 0.0)
            return c

        lax.fori_loop(0, _K // 16, emit, 0)
        pltpu.sync_copy(s_fv.at[pl.ds(0, _K)], tv_hbm.at[pl.ds(b * _K, _K)])
        pltpu.sync_copy(s_ib.at[pl.ds(0, _K)], ti_hbm.at[pl.ds(b * _K, _K)])


def kernel(occ_probs, occ_xyz, b_inds):
    xyz_flat = occ_xyz.reshape(-1)
    occ_flat = occ_probs.reshape(-1)
    vox_flat, lin = _k1(xyz_flat, b_inds.astype(I32))
    hist = _k2(lin)
    tv, ti, occv = _k3(occ_flat)
    vox_coords = vox_flat.reshape(_N, 4)
    top_vals = tv.reshape(_B, _K)
    top_inds = ti.reshape(_B, _K)
    occ_count = jnp.concatenate([occv[0:2], occv[8:10]])
    return vox_coords, hist, top_vals, top_inds, occ_count


# speedup vs baseline: 3.7214x; 1.0680x over previous
import functools

import jax
import jax.numpy as jnp
from jax import lax
from jax.experimental import pallas as pl
from jax.experimental.pallas import tpu as pltpu, tpu_sc as plsc

I32 = jnp.int32
F32 = jnp.float32

_B = 4
_NZ, _NY, _NX = 20, 500, 440
_N = 2_000_000
_NBIN_TOT = _B * _NZ * _NY * _NX
_K = 12000
_PER_BATCH = _NZ * _NY * _NX

_NCH = 1000
_CH = 2000

_SZ = 1_280_000
_NPASS = 7
_TZ = _SZ // 16
_ZCH = 20000

_VBINS = 8192
_KSEL = _K + 1024
_CAPT = 4096
_CBUF = _CAPT + 64
_SCAP = 16384
_SH_CAP = 18432
_CH3 = 4400

_NBLK = 63 * _NZ

_mesh = plsc.VectorSubcoreMesh(core_axis_name="c", subcore_axis_name="s")
_params = pltpu.CompilerParams(needs_layout_passes=False)
_params_t = pltpu.CompilerParams(
    needs_layout_passes=False, use_tc_tiling_on_sc=True)


def _iota():
    return lax.iota(I32, 16)


def _zeros16():
    return lax.full((16,), 0, I32)


def _ones16():
    return lax.full((16,), 1, I32)


def _scal(vec, lane):
    return jnp.sum(jnp.where(_iota() == lane, vec, 0))


@functools.partial(
    pl.kernel,
    out_type=(
        jax.ShapeDtypeStruct((_N * 4,), I32),
        jax.ShapeDtypeStruct((_N,), I32),
    ),
    mesh=_mesh,
    compiler_params=_params,
    scratch_types=[
        pltpu.VMEM((_CH * 3,), F32),
        pltpu.VMEM((_CH,), I32),
        pltpu.VMEM((_CH * 4,), I32),
        pltpu.VMEM((_CH,), I32),
    ],
)
def _k1(xyz_hbm, b_hbm, vox_hbm, lin_hbm, v_in, v_b, v_out, v_lin):
    cid_c = lax.axis_index("c")
    sid = lax.axis_index("s")
    wid = cid_c * 16 + sid
    nch = 31 + jnp.where(wid < _NCH - 31 * 32, 1, 0)

    def chunk(j, carry):
        cid = wid + 32 * j
        base = cid * _CH
        pltpu.sync_copy(xyz_hbm.at[pl.ds(base * 3, _CH * 3)], v_in)
        pltpu.sync_copy(b_hbm.at[pl.ds(base, _CH)], v_b)

        def vec(i, carry2):
            q = i * 16 + _iota()
            ux = plsc.load_gather(v_in, [q * 3])
            uy = plsc.load_gather(v_in, [q * 3 + 1])
            uz = plsc.load_gather(v_in, [q * 3 + 2])
            cx = ((ux * 70.4 + 0.0) - 0.0) / 0.16
            cy = ((uy * 80.0 + -40.0) - -40.0) / 0.16
            cz = ((uz * 4.0 + -3.0) - -3.0) / 0.2
            cxi = jnp.clip(cx.astype(I32), 0, _NX - 1)
            cyi = jnp.clip(cy.astype(I32), 0, _NY - 1)
            czi = jnp.clip(cz.astype(I32), 0, _NZ - 1)
            bb = v_b[pl.ds(i * 16, 16)]
            lin = ((bb * _NZ + czi) * _NY + cyi) * _NX + cxi
            plsc.store_scatter(v_out, [q * 4], bb)
            plsc.store_scatter(v_out, [q * 4 + 1], czi)
            plsc.store_scatter(v_out, [q * 4 + 2], cyi)
            plsc.store_scatter(v_out, [q * 4 + 3], cxi)
            v_lin[pl.ds(i * 16, 16)] = lin
            return carry2

        lax.fori_loop(0, _CH // 16, vec, 0)
        pltpu.sync_copy(v_out, vox_hbm.at[pl.ds(base * 4, _CH * 4)])
        pltpu.sync_copy(v_lin, lin_hbm.at[pl.ds(base, _CH)])
        return carry

    lax.fori_loop(0, nch, chunk, 0)


@functools.partial(
    pl.kernel,
    out_type=jax.ShapeDtypeStruct((_NBIN_TOT,), I32),
    mesh=_mesh,
    compiler_params=_params,
    scratch_types=[
        pltpu.VMEM((_CH,), I32),
        pltpu.VMEM((_CH,), I32),
        pltpu.VMEM((_CH,), I32),
        pltpu.VMEM((_ZCH,), I32),
        pltpu.VMEM((_ZCH,), I32),
        pltpu.VMEM_SHARED((_SZ,), I32),
    ],
)
def _k2(lin_hbm, hist_hbm, v_lin, v_idx, v_one, v_zero, v_obuf, sh_bins):
    cid_c = lax.axis_index("c")
    sid = lax.axis_index("s")
    nch = 62 + jnp.where(sid < _NCH - 62 * 16, 1, 0)

    def fill(i, c):
        v_one[pl.ds(i * 16, 16)] = _ones16()
        return c

    lax.fori_loop(0, _CH // 16, fill, 0)

    def fillz(i, c):
        v_zero[pl.ds(i * 16, 16)] = _zeros16()
        return c

    lax.fori_loop(0, _ZCH // 16, fillz, 0)

    for p in range(_NPASS):
        base = (2 * p + cid_c) * _SZ
        for t in range(_TZ // _ZCH):
            pltpu.sync_copy(
                v_zero, sh_bins.at[pl.ds(sid * _TZ + t * _ZCH, _ZCH)])
        plsc.subcore_barrier()

        def chunk(j, carry):
            cid = sid + 16 * j
            pltpu.sync_copy(lin_hbm.at[pl.ds(cid * _CH, _CH)], v_lin)

            def vec(i, c2):
                loc = v_lin[pl.ds(i * 16, 16)] - base
                inb = (loc >= 0) & (loc < _SZ)
                v_idx[pl.ds(i * 16, 16)] = jnp.where(inb, loc, -1)
                return c2

            lax.fori_loop(0, _CH // 16, vec, 0)
            pltpu.sync_copy(
                v_one, sh_bins.at[plsc.Indices(v_idx, ignored_value=-1)],
                add=True)
            return carry

        lax.fori_loop(0, nch, chunk, 0)
        plsc.subcore_barrier()
        for t in range(_TZ // _ZCH):
            off = sid * _TZ + t * _ZCH

            @pl.when(base + off + _ZCH <= _NBIN_TOT)
            def _():
                pltpu.sync_copy(sh_bins.at[pl.ds(off, _ZCH)], v_obuf)
                pltpu.sync_copy(
                    v_obuf, hist_hbm.at[pl.ds(base + off, _ZCH)])

        plsc.subcore_barrier()


@functools.partial(
    pl.kernel,
    out_type=(
        jax.ShapeDtypeStruct((_B * _K,), F32),
        jax.ShapeDtypeStruct((_B * _K,), I32),
        jax.ShapeDtypeStruct((16,), I32),
    ),
    mesh=_mesh,
    compiler_params=_params_t,
    scratch_types=[
        pltpu.VMEM((8, 440), F32),
        pltpu.VMEM((_VBINS,), I32),
        pltpu.VMEM((_VBINS,), I32),
        pltpu.VMEM((_CBUF,), F32),
        pltpu.VMEM((_CBUF,), I32),
        pltpu.VMEM((16,), I32),
        pltpu.VMEM((_SCAP,), F32),
        pltpu.VMEM((_SCAP,), I32),
        pltpu.VMEM((_SCAP,), I32),
        pltpu.VMEM((_SCAP,), I32),
        pltpu.VMEM((_SCAP,), I32),
        pltpu.VMEM((1024,), I32),
        pltpu.VMEM((1024,), I32),
        pltpu.VMEM_SHARED((16 * _VBINS,), I32),
        pltpu.VMEM_SHARED((2 * _SH_CAP,), F32),
        pltpu.VMEM_SHARED((2 * _SH_CAP,), I32),
        pltpu.VMEM_SHARED((48 * 16,), I32),
    ],
)
def _k3(occ_hbm, tv_hbm, ti_hbm, occ_hbm_out,
        v_buf, v_hist, v_acc, v_cv, v_ci, v_meta,
        s_fv, s_ka, s_ia, s_kb, s_ib, s_dh, s_do,
        sh_hist, sh_v, sh_i, sh_meta):
    cid_c = lax.axis_index("c")
    sid = lax.axis_index("s")
    lb = sid // 8
    m = sid % 8
    b = 2 * cid_c + lb
    nblk = 157 + jnp.where(m < 4, 1, 0)
    blk0 = m * 157 + jnp.minimum(m, 4)
    io = _iota()

    def _fetch(z, yb):
        @pl.when(yb < 62)
        def _():
            pltpu.sync_copy(
                occ_hbm.at[b, z, pl.ds(yb * 8, 8), pl.ds(0, 440)], v_buf)

        @pl.when(yb == 62)
        def _():
            pltpu.sync_copy(occ_hbm.at[b, z, pl.ds(496, 4), pl.ds(0, 440)],
                            v_buf.at[pl.ds(0, 4)])

    def zh(i, c):
        v_hist[pl.ds(i * 16, 16)] = _zeros16()
        return c

    lax.fori_loop(0, _VBINS // 16, zh, 0)

    def p1blk(t, cntv):
        g = blk0 + t
        z = g // 63
        yb = g % 63
        _fetch(z, yb)
        nrows = jnp.where(yb < 62, 8, 4)

        def row(r, cv):
            def col(ci, cv2):
                v = v_buf[r, pl.ds(ci * 16, 16)]
                bn = jnp.minimum((v * float(_VBINS)).astype(I32), _VBINS - 1)
                plsc.addupdate_scatter(v_hist, [bn], _ones16())
                return cv2 + jnp.where(v > 0.5, 1, 0)

            cv = lax.fori_loop(0, 27, col, cv)
            vt = v_buf[r, pl.ds(424, 16)]
            bnt = jnp.minimum((vt * float(_VBINS)).astype(I32), _VBINS - 1)
            mt = io >= 8
            plsc.addupdate_scatter(v_hist, [bnt], _ones16(), mask=mt)
            return cv + jnp.where(mt & (vt > 0.5), 1, 0)

        return lax.fori_loop(0, nrows, row, cntv)

    cntv = lax.fori_loop(0, nblk, p1blk, _zeros16())
    pltpu.sync_copy(v_hist, sh_hist.at[pl.ds(sid * _VBINS, _VBINS)])
    v_meta[...] = cntv
    pltpu.sync_copy(v_meta, sh_meta.at[pl.ds(sid * 16, 16)])
    plsc.subcore_barrier()

    @pl.when(m == 0)
    def _():
        def za(i, c):
            v_acc[pl.ds(i * 16, 16)] = _zeros16()
            return c

        lax.fori_loop(0, _VBINS // 16, za, 0)
        occv = _zeros16()
        for r in range(8):
            pltpu.sync_copy(sh_hist.at[pl.ds((lb * 8 + r) * _VBINS, _VBINS)], v_hist)

            def addh(i, c):
                v_acc[pl.ds(i * 16, 16)] = (
                    v_acc[pl.ds(i * 16, 16)] + v_hist[pl.ds(i * 16, 16)])
                return c

            lax.fori_loop(0, _VBINS // 16, addh, 0)
            pltpu.sync_copy(sh_meta.at[pl.ds((lb * 8 + r) * 16, 16)], v_meta)
            occv = occv + v_meta[...]
        occ_b = jnp.sum(occv)

        def scan(j, carry):
            total, cbin, found = carry
            jj = _VBINS // 16 - 1 - j
            h = v_acc[pl.ds(jj * 16, 16)]
            sfx = lax.rev(plsc.cumsum(lax.rev(h, (0,))), (0,))
            s_incl = total + sfx
            good = (s_incl >= _KSEL).astype(I32)
            ngood = jnp.sum(good)
            cand = jj * 16 + ngood - 1
            cbin = jnp.where(found == 0, jnp.where(ngood > 0, cand, cbin), cbin)
            found = jnp.where(ngood > 0, 1, found)
            return total + jnp.sum(h), cbin, found

        _, cbin, _ = lax.fori_loop(0, _VBINS // 16, scan, (0, 0, 0))
        v_meta[...] = jnp.where(io == 0, cbin, 0) + jnp.where(io == 1, occ_b, 0)
        pltpu.sync_copy(v_meta, sh_meta.at[pl.ds((32 + lb) * 16, 16)])

    plsc.subcore_barrier()

    @pl.when(sid == 0)
    def _():
        pltpu.sync_copy(sh_meta.at[pl.ds(32 * 16, 16)], v_meta)
        o0 = _scal(v_meta[...], 1)
        pltpu.sync_copy(sh_meta.at[pl.ds(33 * 16, 16)], v_meta)
        o1 = _scal(v_meta[...], 1)
        v_meta[...] = jnp.where(io == 0, o0, 0) + jnp.where(io == 1, o1, 0)
        pltpu.sync_copy(v_meta.at[pl.ds(0, 8)],
                        occ_hbm_out.at[pl.ds(8 * cid_c, 8)])

    pltpu.sync_copy(sh_meta.at[pl.ds((32 + lb) * 16, 16)], v_meta)
    cbin = _scal(v_meta[...], 0)

    def p3blk(t, off):
        g = blk0 + t
        z = g // 63
        yb = g % 63
        _fetch(z, yb)
        nrows = jnp.where(yb < 62, 8, 4)

        def row(r, off_r):
            gbase = (z * _NY + yb * 8 + r) * _NX

            def col(ci, off2):
                v = v_buf[r, pl.ds(ci * 16, 16)]
                bn = jnp.minimum((v * float(_VBINS)).astype(I32), _VBINS - 1)
                msk = bn >= cbin
                gidx = gbase + ci * 16 + io

                @pl.when(off2 <= _CAPT)
                def _():
                    plsc.store_compressed(v_cv.at[pl.ds(off2, 16)], v,
                                          mask=msk)
                    plsc.store_compressed(v_ci.at[pl.ds(off2, 16)], gidx,
                                          mask=msk)

                return jnp.minimum(off2 + jnp.sum(msk.astype(I32)),
                                   _CAPT + 16)

            off_r = lax.fori_loop(0, 27, col, off_r)
            vt = v_buf[r, pl.ds(424, 16)]
            bnt = jnp.minimum((vt * float(_VBINS)).astype(I32), _VBINS - 1)
            mskt = (io >= 8) & (bnt >= cbin)
            gidxt = gbase + 424 + io

            @pl.when(off_r <= _CAPT)
            def _():
                plsc.store_compressed(v_cv.at[pl.ds(off_r, 16)], vt,
                                      mask=mskt)
                plsc.store_compressed(v_ci.at[pl.ds(off_r, 16)], gidxt,
                                      mask=mskt)

            return jnp.minimum(off_r + jnp.sum(mskt.astype(I32)), _CAPT + 16)

        return lax.fori_loop(0, nrows, row, off)

    off = lax.fori_loop(0, nblk, p3blk, 0)
    npad = (-off) & 15
    padmask = io < npad
    plsc.store_compressed(v_cv.at[pl.ds(off, 16)],
                          lax.full((16,), -1.0, F32), mask=padmask)
    plsc.store_compressed(v_ci.at[pl.ds(off, 16)], _zeros16(), mask=padmask)
    offp = off + npad
    v_meta[...] = jnp.where(io == 0, off, 0) + jnp.where(io == 1, offp, 0)
    pltpu.sync_copy(v_meta, sh_meta.at[pl.ds((16 + sid) * 16, 16)])
    plsc.subcore_barrier()

    pref = 0
    n_real = 0
    n_pad = 0
    for r in range(8):
        pltpu.sync_copy(sh_meta.at[pl.ds((16 + lb * 8 + r) * 16, 16)], v_meta)
        cr = _scal(v_meta[...], 0)
        cp = _scal(v_meta[...], 1)
        pref = pref + jnp.where(r < m, cp, 0)
        n_real = n_real + cr
        n_pad = n_pad + cp
    mine = offp

    def pub(t, c):
        dst = pl.multiple_of(pref + t * 16, 16)

        @pl.when(dst <= _SH_CAP - 16)
        def _():
            pltpu.sync_copy(v_cv.at[pl.ds(t * 16, 16)],
                            sh_v.at[pl.ds(lb * _SH_CAP + dst, 16)])
            pltpu.sync_copy(v_ci.at[pl.ds(t * 16, 16)],
                            sh_i.at[pl.ds(lb * _SH_CAP + dst, 16)])

        return c

    lax.fori_loop(0, mine // 16, pub, 0)
    plsc.subcore_barrier()

    @pl.when(m == 0)
    def _():
        n_eff = jnp.minimum(n_pad, _SCAP)
        nv = n_eff // 16
        pltpu.sync_copy(sh_v.at[pl.ds(lb * _SH_CAP, _SCAP)], s_fv)
        pltpu.sync_copy(sh_i.at[pl.ds(lb * _SH_CAP, _SCAP)], s_ia)

        def keys(i, c):
            v = s_fv[pl.ds(i * 16, 16)]
            bits = plsc.bitcast(v, I32)
            key = jnp.where(bits < 0, 0x3FFFFFFF, 0x3FFFFFFE - bits)
            s_ka[pl.ds(i * 16, 16)] = key
            return c

        lax.fori_loop(0, nv, keys, 0)

        def radix_pass(shift, src_k, src_i, dst_k, dst_i):
            def zd(i, c):
                s_dh[pl.ds(i * 16, 16)] = _zeros16()
                return c

            lax.fori_loop(0, 64, zd, 0)

            def hist(i, c):
                d = (lax.shift_right_logical(src_k[pl.ds(i * 16, 16)], shift)
                     & 1023)
                cnt, last = plsc.scan_count(d)
                plsc.addupdate_scatter(s_dh, [d], cnt, mask=last)
                return c

            lax.fori_loop(0, nv, hist, 0)

            def pfx(i, tot):
                h = s_dh[pl.ds(i * 16, 16)]
                cs = plsc.cumsum(h)
                s_do[pl.ds(i * 16, 16)] = tot + cs - h
                return tot + jnp.sum(h)

            lax.fori_loop(0, 64, pfx, 0)

            def scat(i, c):
                k = src_k[pl.ds(i * 16, 16)]
                pv = src_i[pl.ds(i * 16, 16)]
                d = lax.shift_right_logical(k, shift) & 1023
                cnt, last = plsc.scan_count(d)
                pos = plsc.load_gather(s_do, [d]) + cnt - 1
                plsc.store_scatter(dst_k, [pos], k)
                plsc.store_scatter(dst_i, [pos], pv)
                plsc.addupdate_scatter(s_do, [d], cnt, mask=last)
                return c

            lax.fori_loop(0, nv, scat, 0)

        radix_pass(0, s_ka, s_ia, s_kb, s_ib)
        radix_pass(10, s_kb, s_ib, s_ka, s_ia)
        radix_pass(20, s_ka, s_ia, s_kb, s_ib)

        def emit(i, c):
            key = s_kb[pl.ds(i * 16, 16)]
            bits = 0x3FFFFFFE - key
            v = plsc.bitcast(bits, F32)
            s_fv[pl.ds(i * 16, 16)] = jnp.where(v > 0.5, v, 0.0)
            return c

        lax.fori_loop(0, _K // 16, emit, 0)
        pltpu.sync_copy(s_fv.at[pl.ds(0, _K)], tv_hbm.at[pl.ds(b * _K, _K)])
        pltpu.sync_copy(s_ib.at[pl.ds(0, _K)], ti_hbm.at[pl.ds(b * _K, _K)])


def kernel(occ_probs, occ_xyz, b_inds):
    xyz_flat = occ_xyz.reshape(-1)
    vox_flat, lin = _k1(xyz_flat, b_inds.astype(I32))
    hist = _k2(lin)
    tv, ti, occv = _k3(occ_probs)
    vox_coords = vox_flat.reshape(_N, 4)
    top_vals = tv.reshape(_B, _K)
    top_inds = ti.reshape(_B, _K)
    occ_count = jnp.concatenate([occv[0:2], occv[8:10]])
    return vox_coords, hist, top_vals, top_inds, occ_count


# speedup vs baseline: 12.3304x; 3.3134x over previous
import functools

import jax
import jax.numpy as jnp
from jax import lax
from jax.experimental import pallas as pl
from jax.experimental.pallas import tpu as pltpu, tpu_sc as plsc

I32 = jnp.int32
F32 = jnp.float32

_B = 4
_NZ, _NY, _NX = 20, 500, 440
_N = 2_000_000
_NBIN_TOT = _B * _NZ * _NY * _NX
_K = 12000
_PER_BATCH = _NZ * _NY * _NX

_NCH = 1000
_CH = 2000

_SZ = 1_280_000
_NPASS = 7
_TZ = _SZ // 16
_ZCH = 20000

_VBINS = 8192
_KSEL = _K + 1024
_CAPT = 4096
_CBUF = _CAPT + 64
_SCAP = 16384
_SH_CAP = 18432
_CH3 = 4400

_NBLK = 63 * _NZ

_mesh = plsc.VectorSubcoreMesh(core_axis_name="c", subcore_axis_name="s")
_params = pltpu.CompilerParams(needs_layout_passes=False)
_params_t = pltpu.CompilerParams(
    needs_layout_passes=False, use_tc_tiling_on_sc=True)


def _iota():
    return lax.iota(I32, 16)


def _zeros16():
    return lax.full((16,), 0, I32)


def _ones16():
    return lax.full((16,), 1, I32)


def _scal(vec, lane):
    return jnp.sum(jnp.where(_iota() == lane, vec, 0))


_C1 = 2048
_NC1 = 977
_TAIL1 = _N - 976 * _C1


@functools.partial(
    pl.kernel,
    out_type=(
        jax.ShapeDtypeStruct((_N * 4,), I32),
        jax.ShapeDtypeStruct((_N,), I32),
    ),
    mesh=_mesh,
    compiler_params=_params_t,
    scratch_types=[
        pltpu.VMEM((3, _C1), F32),
        pltpu.VMEM((_C1,), I32),
        pltpu.VMEM((_C1 * 4,), I32),
        pltpu.VMEM((_C1,), I32),
    ],
)
def _k1(xyz_hbm, b_hbm, vox_hbm, lin_hbm, v_in, v_b, v_out, v_lin):
    cid_c = lax.axis_index("c")
    sid = lax.axis_index("s")
    wid = cid_c * 16 + sid
    nch = 30 + jnp.where(wid < _NC1 - 30 * 32, 1, 0)

    def chunk(j, carry):
        cid = wid + 32 * j
        base = cid * _C1
        tail = cid == _NC1 - 1

        @pl.when(~tail)
        def _():
            pltpu.sync_copy(
                xyz_hbm.at[pl.ds(0, 3), pl.ds(base, _C1)], v_in)
            pltpu.sync_copy(b_hbm.at[pl.ds(base, _C1)], v_b)

        @pl.when(tail)
        def _():
            pltpu.sync_copy(
                xyz_hbm.at[pl.ds(0, 3), pl.ds(base, _TAIL1)],
                v_in.at[pl.ds(0, 3), pl.ds(0, _TAIL1)])
            pltpu.sync_copy(b_hbm.at[pl.ds(base, _TAIL1)],
                            v_b.at[pl.ds(0, _TAIL1)])

        nv = jnp.where(tail, _TAIL1 // 16, _C1 // 16)

        def vec(i, carry2):
            q = i * 16 + _iota()
            ux = v_in[0, pl.ds(i * 16, 16)]
            uy = v_in[1, pl.ds(i * 16, 16)]
            uz = v_in[2, pl.ds(i * 16, 16)]
            cx = ((ux * 70.4 + 0.0) - 0.0) / 0.16
            cy = ((uy * 80.0 + -40.0) - -40.0) / 0.16
            cz = ((uz * 4.0 + -3.0) - -3.0) / 0.2
            cxi = jnp.clip(cx.astype(I32), 0, _NX - 1)
            cyi = jnp.clip(cy.astype(I32), 0, _NY - 1)
            czi = jnp.clip(cz.astype(I32), 0, _NZ - 1)
            bb = v_b[pl.ds(i * 16, 16)]
            lin = ((bb * _NZ + czi) * _NY + cyi) * _NX + cxi
            plsc.store_scatter(v_out, [q * 4], bb)
            plsc.store_scatter(v_out, [q * 4 + 1], czi)
            plsc.store_scatter(v_out, [q * 4 + 2], cyi)
            plsc.store_scatter(v_out, [q * 4 + 3], cxi)
            v_lin[pl.ds(i * 16, 16)] = lin
            return carry2

        lax.fori_loop(0, nv, vec, 0)

        @pl.when(~tail)
        def _():
            pltpu.sync_copy(v_out, vox_hbm.at[pl.ds(base * 4, _C1 * 4)])
            pltpu.sync_copy(v_lin, lin_hbm.at[pl.ds(base, _C1)])

        @pl.when(tail)
        def _():
            pltpu.sync_copy(v_out.at[pl.ds(0, _TAIL1 * 4)],
                            vox_hbm.at[pl.ds(base * 4, _TAIL1 * 4)])
            pltpu.sync_copy(v_lin.at[pl.ds(0, _TAIL1)],
                            lin_hbm.at[pl.ds(base, _TAIL1)])

        return carry

    lax.fori_loop(0, nch, chunk, 0)


@functools.partial(
    pl.kernel,
    out_type=jax.ShapeDtypeStruct((_NBIN_TOT,), I32),
    mesh=_mesh,
    compiler_params=_params,
    scratch_types=[
        pltpu.VMEM((_CH,), I32),
        pltpu.VMEM((_CH,), I32),
        pltpu.VMEM((_CH,), I32),
        pltpu.VMEM((_ZCH,), I32),
        pltpu.VMEM((_ZCH,), I32),
        pltpu.VMEM_SHARED((_SZ,), I32),
    ],
)
def _k2(lin_hbm, hist_hbm, v_lin, v_idx, v_one, v_zero, v_obuf, sh_bins):
    cid_c = lax.axis_index("c")
    sid = lax.axis_index("s")
    nch = 62 + jnp.where(sid < _NCH - 62 * 16, 1, 0)

    def fill(i, c):
        v_one[pl.ds(i * 16, 16)] = _ones16()
        return c

    lax.fori_loop(0, _CH // 16, fill, 0)

    def fillz(i, c):
        v_zero[pl.ds(i * 16, 16)] = _zeros16()
        return c

    lax.fori_loop(0, _ZCH // 16, fillz, 0)

    for p in range(_NPASS):
        base = (2 * p + cid_c) * _SZ
        for t in range(_TZ // _ZCH):
            pltpu.sync_copy(
                v_zero, sh_bins.at[pl.ds(sid * _TZ + t * _ZCH, _ZCH)])
        plsc.subcore_barrier()

        def chunk(j, carry):
            cid = sid + 16 * j
            pltpu.sync_copy(lin_hbm.at[pl.ds(cid * _CH, _CH)], v_lin)

            def vec(i, c2):
                loc = v_lin[pl.ds(i * 16, 16)] - base
                inb = (loc >= 0) & (loc < _SZ)
                v_idx[pl.ds(i * 16, 16)] = jnp.where(inb, loc, -1)
                return c2

            lax.fori_loop(0, _CH // 16, vec, 0)
            pltpu.sync_copy(
                v_one, sh_bins.at[plsc.Indices(v_idx, ignored_value=-1)],
                add=True)
            return carry

        lax.fori_loop(0, nch, chunk, 0)
        plsc.subcore_barrier()
        for t in range(_TZ // _ZCH):
            off = sid * _TZ + t * _ZCH

            @pl.when(base + off + _ZCH <= _NBIN_TOT)
            def _():
                pltpu.sync_copy(sh_bins.at[pl.ds(off, _ZCH)], v_obuf)
                pltpu.sync_copy(
                    v_obuf, hist_hbm.at[pl.ds(base + off, _ZCH)])

        plsc.subcore_barrier()


@functools.partial(
    pl.kernel,
    out_type=(
        jax.ShapeDtypeStruct((_B * _K,), F32),
        jax.ShapeDtypeStruct((_B * _K,), I32),
        jax.ShapeDtypeStruct((16,), I32),
    ),
    mesh=_mesh,
    compiler_params=_params_t,
    scratch_types=[
        pltpu.VMEM((8, 440), F32),
        pltpu.VMEM((_VBINS,), I32),
        pltpu.VMEM((_VBINS,), I32),
        pltpu.VMEM((_CBUF,), F32),
        pltpu.VMEM((_CBUF,), I32),
        pltpu.VMEM((16,), I32),
        pltpu.VMEM((_SCAP,), F32),
        pltpu.VMEM((_SCAP,), I32),
        pltpu.VMEM((_SCAP,), I32),
        pltpu.VMEM((_SCAP,), I32),
        pltpu.VMEM((_SCAP,), I32),
        pltpu.VMEM((1024,), I32),
        pltpu.VMEM((1024,), I32),
        pltpu.VMEM_SHARED((16 * _VBINS,), I32),
        pltpu.VMEM_SHARED((2 * _SH_CAP,), F32),
        pltpu.VMEM_SHARED((2 * _SH_CAP,), I32),
        pltpu.VMEM_SHARED((48 * 16,), I32),
    ],
)
def _k3(occ_hbm, tv_hbm, ti_hbm, occ_hbm_out,
        v_buf, v_hist, v_acc, v_cv, v_ci, v_meta,
        s_fv, s_ka, s_ia, s_kb, s_ib, s_dh, s_do,
        sh_hist, sh_v, sh_i, sh_meta):
    cid_c = lax.axis_index("c")
    sid = lax.axis_index("s")
    lb = sid // 8
    m = sid % 8
    b = 2 * cid_c + lb
    nblk = 157 + jnp.where(m < 4, 1, 0)
    blk0 = m * 157 + jnp.minimum(m, 4)
    io = _iota()

    def _fetch(z, yb):
        @pl.when(yb < 62)
        def _():
            pltpu.sync_copy(
                occ_hbm.at[b, z, pl.ds(yb * 8, 8), pl.ds(0, 440)], v_buf)

        @pl.when(yb == 62)
        def _():
            pltpu.sync_copy(occ_hbm.at[b, z, pl.ds(496, 4), pl.ds(0, 440)],
                            v_buf.at[pl.ds(0, 4)])

    def zh(i, c):
        v_hist[pl.ds(i * 16, 16)] = _zeros16()
        return c

    lax.fori_loop(0, _VBINS // 16, zh, 0)

    def p1blk(t, cntv):
        g = blk0 + t
        z = g // 63
        yb = g % 63
        _fetch(z, yb)
        nrows = jnp.where(yb < 62, 8, 4)

        def row(r, cv):
            def col(ci, cv2):
                v = v_buf[r, pl.ds(ci * 16, 16)]
                bn = jnp.minimum((v * float(_VBINS)).astype(I32), _VBINS - 1)
                plsc.addupdate_scatter(v_hist, [bn], _ones16())
                return cv2 + jnp.where(v > 0.5, 1, 0)

            cv = lax.fori_loop(0, 27, col, cv)
            vt = v_buf[r, pl.ds(424, 16)]
            bnt = jnp.minimum((vt * float(_VBINS)).astype(I32), _VBINS - 1)
            mt = io >= 8
            plsc.addupdate_scatter(v_hist, [bnt], _ones16(), mask=mt)
            return cv + jnp.where(mt & (vt > 0.5), 1, 0)

        return lax.fori_loop(0, nrows, row, cntv)

    cntv = lax.fori_loop(0, nblk, p1blk, _zeros16())
    pltpu.sync_copy(v_hist, sh_hist.at[pl.ds(sid * _VBINS, _VBINS)])
    v_meta[...] = cntv
    pltpu.sync_copy(v_meta, sh_meta.at[pl.ds(sid * 16, 16)])
    plsc.subcore_barrier()

    @pl.when(m == 0)
    def _():
        def za(i, c):
            v_acc[pl.ds(i * 16, 16)] = _zeros16()
            return c

        lax.fori_loop(0, _VBINS // 16, za, 0)
        occv = _zeros16()
        for r in range(8):
            pltpu.sync_copy(sh_hist.at[pl.ds((lb * 8 + r) * _VBINS, _VBINS)], v_hist)

            def addh(i, c):
                v_acc[pl.ds(i * 16, 16)] = (
                    v_acc[pl.ds(i * 16, 16)] + v_hist[pl.ds(i * 16, 16)])
                return c

            lax.fori_loop(0, _VBINS // 16, addh, 0)
            pltpu.sync_copy(sh_meta.at[pl.ds((lb * 8 + r) * 16, 16)], v_meta)
            occv = occv + v_meta[...]
        occ_b = jnp.sum(occv)

        def scan(j, carry):
            total, cbin, found = carry
            jj = _VBINS // 16 - 1 - j
            h = v_acc[pl.ds(jj * 16, 16)]
            sfx = lax.rev(plsc.cumsum(lax.rev(h, (0,))), (0,))
            s_incl = total + sfx
            good = (s_incl >= _KSEL).astype(I32)
            ngood = jnp.sum(good)
            cand = jj * 16 + ngood - 1
            cbin = jnp.where(found == 0, jnp.where(ngood > 0, cand, cbin), cbin)
            found = jnp.where(ngood > 0, 1, found)
            return total + jnp.sum(h), cbin, found

        _, cbin, _ = lax.fori_loop(0, _VBINS // 16, scan, (0, 0, 0))
        v_meta[...] = jnp.where(io == 0, cbin, 0) + jnp.where(io == 1, occ_b, 0)
        pltpu.sync_copy(v_meta, sh_meta.at[pl.ds((32 + lb) * 16, 16)])

    plsc.subcore_barrier()

    @pl.when(sid == 0)
    def _():
        pltpu.sync_copy(sh_meta.at[pl.ds(32 * 16, 16)], v_meta)
        o0 = _scal(v_meta[...], 1)
        pltpu.sync_copy(sh_meta.at[pl.ds(33 * 16, 16)], v_meta)
        o1 = _scal(v_meta[...], 1)
        v_meta[...] = jnp.where(io == 0, o0, 0) + jnp.where(io == 1, o1, 0)
        pltpu.sync_copy(v_meta.at[pl.ds(0, 8)],
                        occ_hbm_out.at[pl.ds(8 * cid_c, 8)])

    pltpu.sync_copy(sh_meta.at[pl.ds((32 + lb) * 16, 16)], v_meta)
    cbin = _scal(v_meta[...], 0)

    def p3blk(t, off):
        g = blk0 + t
        z = g // 63
        yb = g % 63
        _fetch(z, yb)
        nrows = jnp.where(yb < 62, 8, 4)

        def row(r, off_r):
            gbase = (z * _NY + yb * 8 + r) * _NX

            def col(ci, off2):
                v = v_buf[r, pl.ds(ci * 16, 16)]
                bn = jnp.minimum((v * float(_VBINS)).astype(I32), _VBINS - 1)
                msk = bn >= cbin
                gidx = gbase + ci * 16 + io

                @pl.when(off2 <= _CAPT)
                def _():
                    plsc.store_compressed(v_cv.at[pl.ds(off2, 16)], v,
                                          mask=msk)
                    plsc.store_compressed(v_ci.at[pl.ds(off2, 16)], gidx,
                                          mask=msk)

                return jnp.minimum(off2 + jnp.sum(msk.astype(I32)),
                                   _CAPT + 16)

            off_r = lax.fori_loop(0, 27, col, off_r)
            vt = v_buf[r, pl.ds(424, 16)]
            bnt = jnp.minimum((vt * float(_VBINS)).astype(I32), _VBINS - 1)
            mskt = (io >= 8) & (bnt >= cbin)
            gidxt = gbase + 424 + io

            @pl.when(off_r <= _CAPT)
            def _():
                plsc.store_compressed(v_cv.at[pl.ds(off_r, 16)], vt,
                                      mask=mskt)
                plsc.store_compressed(v_ci.at[pl.ds(off_r, 16)], gidxt,
                                      mask=mskt)

            return jnp.minimum(off_r + jnp.sum(mskt.astype(I32)), _CAPT + 16)

        return lax.fori_loop(0, nrows, row, off)

    off = lax.fori_loop(0, nblk, p3blk, 0)
    npad = (-off) & 15
    padmask = io < npad
    plsc.store_compressed(v_cv.at[pl.ds(off, 16)],
                          lax.full((16,), -1.0, F32), mask=padmask)
    plsc.store_compressed(v_ci.at[pl.ds(off, 16)], _zeros16(), mask=padmask)
    offp = off + npad
    v_meta[...] = jnp.where(io == 0, off, 0) + jnp.where(io == 1, offp, 0)
    pltpu.sync_copy(v_meta, sh_meta.at[pl.ds((16 + sid) * 16, 16)])
    plsc.subcore_barrier()

    pref = 0
    n_real = 0
    n_pad = 0
    for r in range(8):
        pltpu.sync_copy(sh_meta.at[pl.ds((16 + lb * 8 + r) * 16, 16)], v_meta)
        cr = _scal(v_meta[...], 0)
        cp = _scal(v_meta[...], 1)
        pref = pref + jnp.where(r < m, cp, 0)
        n_real = n_real + cr
        n_pad = n_pad + cp
    mine = offp

    def pub(t, c):
        dst = pl.multiple_of(pref + t * 16, 16)

        @pl.when(dst <= _SH_CAP - 16)
        def _():
            pltpu.sync_copy(v_cv.at[pl.ds(t * 16, 16)],
                            sh_v.at[pl.ds(lb * _SH_CAP + dst, 16)])
            pltpu.sync_copy(v_ci.at[pl.ds(t * 16, 16)],
                            sh_i.at[pl.ds(lb * _SH_CAP + dst, 16)])

        return c

    lax.fori_loop(0, mine // 16, pub, 0)
    plsc.subcore_barrier()

    @pl.when(m == 0)
    def _():
        n_eff = jnp.minimum(n_pad, _SCAP)
        nv = n_eff // 16
        pltpu.sync_copy(sh_v.at[pl.ds(lb * _SH_CAP, _SCAP)], s_fv)
        pltpu.sync_copy(sh_i.at[pl.ds(lb * _SH_CAP, _SCAP)], s_ia)

        def keys(i, c):
            v = s_fv[pl.ds(i * 16, 16)]
            bits = plsc.bitcast(v, I32)
            key = jnp.where(bits < 0, 0x3FFFFFFF, 0x3FFFFFFE - bits)
            s_ka[pl.ds(i * 16, 16)] = key
            return c

        lax.fori_loop(0, nv, keys, 0)

        def radix_pass(shift, src_k, src_i, dst_k, dst_i):
            def zd(i, c):
                s_dh[pl.ds(i * 16, 16)] = _zeros16()
                return c

            lax.fori_loop(0, 64, zd, 0)

            def hist(i, c):
                d = (lax.shift_right_logical(src_k[pl.ds(i * 16, 16)], shift)
                     & 1023)
                cnt, last = plsc.scan_count(d)
                plsc.addupdate_scatter(s_dh, [d], cnt, mask=last)
                return c

            lax.fori_loop(0, nv, hist, 0)

            def pfx(i, tot):
                h = s_dh[pl.ds(i * 16, 16)]
                cs = plsc.cumsum(h)
                s_do[pl.ds(i * 16, 16)] = tot + cs - h
                return tot + jnp.sum(h)

            lax.fori_loop(0, 64, pfx, 0)

            def scat(i, c):
                k = src_k[pl.ds(i * 16, 16)]
                pv = src_i[pl.ds(i * 16, 16)]
                d = lax.shift_right_logical(k, shift) & 1023
                cnt, last = plsc.scan_count(d)
                pos = plsc.load_gather(s_do, [d]) + cnt - 1
                plsc.store_scatter(dst_k, [pos], k)
                plsc.store_scatter(dst_i, [pos], pv)
                plsc.addupdate_scatter(s_do, [d], cnt, mask=last)
                return c

            lax.fori_loop(0, nv, scat, 0)

        radix_pass(0, s_ka, s_ia, s_kb, s_ib)
        radix_pass(10, s_kb, s_ib, s_ka, s_ia)
        radix_pass(20, s_ka, s_ia, s_kb, s_ib)

        def emit(i, c):
            key = s_kb[pl.ds(i * 16, 16)]
            bits = 0x3FFFFFFE - key
            v = plsc.bitcast(bits, F32)
            s_fv[pl.ds(i * 16, 16)] = jnp.where(v > 0.5, v, 0.0)
            return c

        lax.fori_loop(0, _K // 16, emit, 0)
        pltpu.sync_copy(s_fv.at[pl.ds(0, _K)], tv_hbm.at[pl.ds(b * _K, _K)])
        pltpu.sync_copy(s_ib.at[pl.ds(0, _K)], ti_hbm.at[pl.ds(b * _K, _K)])


def kernel(occ_probs, occ_xyz, b_inds):
    xyz_t = occ_xyz.T
    vox_flat, lin = _k1(xyz_t, b_inds.astype(I32))
    hist = _k2(lin)
    tv, ti, occv = _k3(occ_probs)
    vox_coords = vox_flat.reshape(_N, 4)
    top_vals = tv.reshape(_B, _K)
    top_inds = ti.reshape(_B, _K)
    occ_count = jnp.concatenate([occv[0:2], occv[8:10]])
    return vox_coords, hist, top_vals, top_inds, occ_count
